# Initial kernel scaffold; baseline (speedup 1.0000x reference)
import jax
import jax.numpy as jnp
from jax.experimental import pallas as pl


def kernel(input_array):
    raise NotImplementedError("write your pallas kernel here")


# speedup vs baseline: 2.0198x; 2.0198x over previous
import functools

import jax
import jax.numpy as jnp
from jax import lax
from jax.experimental import pallas as pl
from jax.experimental.pallas import tpu as pltpu
from jax.experimental.pallas import tpu_sc as plsc

N = 8388608
NC, NS, L = 2, 16, 16
NW = NC * NS
SEG = 1024
NSEG = N // SEG
SPW = NSEG // NW
NCHUNK = SPW // L

_mesh = plsc.VectorSubcoreMesh(core_axis_name="c", subcore_axis_name="s")


@functools.partial(
    pl.kernel,
    mesh=_mesh,
    out_type=jax.ShapeDtypeStruct((NSEG,), jnp.float32),
    compiler_params=pltpu.CompilerParams(needs_layout_passes=False),
    scratch_types=[
        pltpu.VMEM((L, SEG), jnp.float32),
        pltpu.VMEM((SPW,), jnp.float32),
    ],
)
def _seg_sums(x_hbm, sums_hbm, buf, totals_v):
    wid = lax.axis_index("s") * NC + lax.axis_index("c")
    row0 = wid * SPW
    lane_idx = lax.iota(jnp.int32, L)

    def chunk_body(c, carry):
        pltpu.sync_copy(x_hbm.at[pl.ds(row0 + c * L, L), :], buf)

        def t_body(t, acc):
            col = jnp.full((L,), 0, jnp.int32) + t
            return acc + plsc.load_gather(buf, [lane_idx, col])

        acc = lax.fori_loop(0, SEG, t_body, jnp.zeros((L,), jnp.float32))
        totals_v[pl.ds(c * L, L)] = acc
        return carry

    lax.fori_loop(0, NCHUNK, chunk_body, 0)
    pltpu.sync_copy(totals_v, sums_hbm.at[pl.ds(wid * SPW, SPW)])


@functools.partial(
    pl.kernel,
    mesh=_mesh,
    out_type=jax.ShapeDtypeStruct((NSEG, SEG), jnp.float32),
    compiler_params=pltpu.CompilerParams(needs_layout_passes=False),
    scratch_types=[
        pltpu.VMEM((NSEG,), jnp.float32),
        pltpu.VMEM((SPW,), jnp.float32),
        pltpu.VMEM((L, SEG), jnp.float32),
        pltpu.VMEM((L, SEG), jnp.float32),
    ],
)
def _scan_apply(x_hbm, sums_hbm, out_hbm, sums_v, offs_v, ibuf, obuf):
    wid = lax.axis_index("s") * NC + lax.axis_index("c")
    row0 = wid * SPW
    pltpu.sync_copy(sums_hbm, sums_v)

    def pre_body(i, carry):
        return carry + jnp.sum(sums_v[pl.ds(i * L, L)])

    carry = lax.fori_loop(0, wid * NCHUNK, pre_body, jnp.float32(0.0))

    my_first = wid * SPW
    for i in range(NCHUNK):
        v = sums_v[pl.ds(my_first + i * L, L)]
        offs_v[pl.ds(i * L, L)] = plsc.cumsum(v) - v + carry
        carry = carry + jnp.sum(v)

    lane_idx = lax.iota(jnp.int32, L)

    def chunk_body(c, carry2):
        pltpu.sync_copy(x_hbm.at[pl.ds(row0 + c * L, L), :], ibuf)
        off = offs_v[pl.ds(c * L, L)]

        def t_body(t, acc):
            col = jnp.full((L,), 0, jnp.int32) + t
            acc = acc + plsc.load_gather(ibuf, [lane_idx, col])
            plsc.store_scatter(obuf, [lane_idx, col], acc)
            return acc

        lax.fori_loop(0, SEG, t_body, off)
        pltpu.sync_copy(obuf, out_hbm.at[pl.ds(row0 + c * L, L), :])
        return carry2

    lax.fori_loop(0, NCHUNK, chunk_body, 0)


def kernel(input_array):
    x2 = input_array.reshape(NSEG, SEG)
    sums = _seg_sums(x2)
    out = _scan_apply(x2, sums)
    return out.reshape(N)


# speedup vs baseline: 4.1810x; 2.0700x over previous
import functools

import jax
import jax.numpy as jnp
from jax import lax
from jax.experimental import pallas as pl
from jax.experimental.pallas import tpu as pltpu
from jax.experimental.pallas import tpu_sc as plsc

N = 8388608
NC, NS, L = 2, 16, 16
NW = NC * NS
NPW = N // NW
SEG = 256
GRP = 4
CHUNK = GRP * L * SEG
NCHUNK = NPW // CHUNK
SPC = GRP * L
SPW = NCHUNK * SPC
NSEG = NW * SPW
UNROLL = 4

_mesh = plsc.VectorSubcoreMesh(core_axis_name="c", subcore_axis_name="s")
_params = pltpu.CompilerParams(needs_layout_passes=False)


@functools.partial(
    pl.kernel,
    mesh=_mesh,
    out_type=(jax.ShapeDtypeStruct((NSEG,), jnp.float32),
              jax.ShapeDtypeStruct((NW * L,), jnp.float32)),
    compiler_params=_params,
    scratch_types=[
        pltpu.VMEM((CHUNK,), jnp.float32),
        pltpu.VMEM((SPW,), jnp.float32),
        pltpu.VMEM((L,), jnp.float32),
    ],
)
def _seg_sums(x_hbm, sums_hbm, wsums_hbm, buf, totals_v, wtot_v):
    wid = lax.axis_index("s") * NC + lax.axis_index("c")
    base = wid * NPW
    idx0 = tuple(lax.iota(jnp.int32, L) * SEG + g * (L * SEG)
                 for g in range(GRP))

    def chunk_body(c, carry):
        pltpu.sync_copy(x_hbm.at[pl.ds(base + c * CHUNK, CHUNK)], buf)

        @plsc.parallel_loop(
            0, SEG, unroll=UNROLL,
            carry=(tuple(jnp.zeros((L,), jnp.float32) for _ in range(GRP)),
                   idx0))
        def t_body(t, ai):
            accs, idxs = ai
            accs = tuple(a + plsc.load_gather(buf, [i])
                         for a, i in zip(accs, idxs))
            return accs, tuple(i + 1 for i in idxs)

        accs, _ = t_body
        for g in range(GRP):
            totals_v[pl.ds(c * SPC + g * L, L)] = accs[g]
        return carry

    lax.fori_loop(0, NCHUNK, chunk_body, 0)

    def tot_body(i, acc):
        return acc + totals_v[pl.ds(i * L, L)]

    tot = lax.fori_loop(0, SPW // L, tot_body, jnp.zeros((L,), jnp.float32),
                        unroll=4)
    wtot_v[...] = jnp.zeros((L,), jnp.float32) + jnp.sum(tot)
    pltpu.sync_copy(totals_v, sums_hbm.at[pl.ds(wid * SPW, SPW)])
    pltpu.sync_copy(wtot_v, wsums_hbm.at[pl.ds(wid * L, L)])


@functools.partial(
    pl.kernel,
    mesh=_mesh,
    out_type=jax.ShapeDtypeStruct((N,), jnp.float32),
    compiler_params=_params,
    scratch_types=[
        pltpu.VMEM((NW * L,), jnp.float32),
        pltpu.VMEM((SPW,), jnp.float32),
        pltpu.VMEM((SPW,), jnp.float32),
        pltpu.VMEM((CHUNK,), jnp.float32),
    ],
)
def _scan_apply(x_hbm, sums_hbm, wsums_hbm, out_hbm,
                wsums_v, sums_v, offs_v, buf):
    wid = lax.axis_index("s") * NC + lax.axis_index("c")
    base = wid * NPW
    pltpu.sync_copy(wsums_hbm, wsums_v)
    pltpu.sync_copy(sums_hbm.at[pl.ds(wid * SPW, SPW)], sums_v)

    def pre_body(w, cv):
        return cv + plsc.load_gather(wsums_v, [jnp.full((L,), 0, jnp.int32) + w * L])

    carry = lax.fori_loop(0, wid, pre_body, jnp.zeros((L,), jnp.float32))

    def off_body(i, cv):
        v = sums_v[pl.ds(i * L, L)]
        offs_v[pl.ds(i * L, L)] = plsc.cumsum(v) - v + cv
        return cv + jnp.sum(v)

    lax.fori_loop(0, SPW // L, off_body, carry, unroll=4)

    idx0 = tuple(lax.iota(jnp.int32, L) * SEG + g * (L * SEG)
                 for g in range(GRP))

    def chunk_body(c, carry2):
        pltpu.sync_copy(x_hbm.at[pl.ds(base + c * CHUNK, CHUNK)], buf)
        offs = tuple(offs_v[pl.ds(c * SPC + g * L, L)] for g in range(GRP))

        @plsc.parallel_loop(0, SEG, unroll=UNROLL, carry=(offs, idx0))
        def t_body(t, ai):
            accs, idxs = ai
            new_accs = []
            for g in range(GRP):
                a = accs[g] + plsc.load_gather(buf, [idxs[g]])
                plsc.store_scatter(buf, [idxs[g]], a)
                new_accs.append(a)
            return tuple(new_accs), tuple(i + 1 for i in idxs)

        del t_body
        pltpu.sync_copy(buf, out_hbm.at[pl.ds(base + c * CHUNK, CHUNK)])
        return carry2

    lax.fori_loop(0, NCHUNK, chunk_body, 0)


def kernel(input_array):
    sums, wsums = _seg_sums(input_array)
    return _scan_apply(input_array, sums, wsums)


# speedup vs baseline: 4.7623x; 1.1390x over previous
import functools

import jax
import jax.numpy as jnp
from jax import lax
from jax.experimental import pallas as pl
from jax.experimental.pallas import tpu as pltpu
from jax.experimental.pallas import tpu_sc as plsc

N = 8388608
NC, NS, L = 2, 16, 16
NW = NC * NS
NPW = N // NW
SEG = 512
GRP = 4
CHUNK = GRP * L * SEG
NCHUNK = NPW // CHUNK
SPC = GRP * L
SPW = NCHUNK * SPC
NSEG = NW * SPW
UNROLL = 4

_mesh = plsc.VectorSubcoreMesh(core_axis_name="c", subcore_axis_name="s")
_params = pltpu.CompilerParams(needs_layout_passes=False)


def _idx0():
    return tuple(lax.iota(jnp.int32, L) * SEG + g * (L * SEG)
                 for g in range(GRP))


@functools.partial(
    pl.kernel,
    mesh=_mesh,
    out_type=(jax.ShapeDtypeStruct((NSEG,), jnp.float32),
              jax.ShapeDtypeStruct((NW * L,), jnp.float32)),
    compiler_params=_params,
    scratch_types=[
        pltpu.VMEM((CHUNK,), jnp.float32),
        pltpu.VMEM((CHUNK,), jnp.float32),
        pltpu.VMEM((SPW,), jnp.float32),
        pltpu.VMEM((L,), jnp.float32),
        pltpu.SemaphoreType.DMA,
        pltpu.SemaphoreType.DMA,
    ],
)
def _seg_sums(x_hbm, sums_hbm, wsums_hbm, buf0, buf1, totals_v, wtot_v,
              sem0, sem1):
    wid = lax.axis_index("s") * NC + lax.axis_index("c")
    base = wid * NPW
    bufs = (buf0, buf1)
    sems = (sem0, sem1)
    idx0 = _idx0()

    handles = [None, None]
    handles[0] = pltpu.async_copy(x_hbm.at[pl.ds(base, CHUNK)], buf0, sem0)
    for c in range(NCHUNK):
        b = c % 2
        if c + 1 < NCHUNK:
            handles[1 - b] = pltpu.async_copy(
                x_hbm.at[pl.ds(base + (c + 1) * CHUNK, CHUNK)],
                bufs[1 - b], sems[1 - b])
        handles[b].wait()
        buf = bufs[b]

        @plsc.parallel_loop(
            0, SEG, unroll=UNROLL,
            carry=(tuple(jnp.zeros((L,), jnp.float32) for _ in range(GRP)),
                   idx0))
        def t_body(t, ai):
            accs, idxs = ai
            accs = tuple(a + plsc.load_gather(buf, [i])
                         for a, i in zip(accs, idxs))
            return accs, tuple(i + 1 for i in idxs)

        accs, _ = t_body
        for g in range(GRP):
            totals_v[pl.ds(c * SPC + g * L, L)] = accs[g]

    def tot_body(i, acc):
        return acc + totals_v[pl.ds(i * L, L)]

    tot = lax.fori_loop(0, SPW // L, tot_body, jnp.zeros((L,), jnp.float32),
                        unroll=4)
    wtot_v[...] = jnp.zeros((L,), jnp.float32) + jnp.sum(tot)
    pltpu.sync_copy(totals_v, sums_hbm.at[pl.ds(wid * SPW, SPW)])
    pltpu.sync_copy(wtot_v, wsums_hbm.at[pl.ds(wid * L, L)])


@functools.partial(
    pl.kernel,
    mesh=_mesh,
    out_type=jax.ShapeDtypeStruct((N,), jnp.float32),
    compiler_params=_params,
    scratch_types=[
        pltpu.VMEM((NW * L,), jnp.float32),
        pltpu.VMEM((SPW,), jnp.float32),
        pltpu.VMEM((SPW,), jnp.float32),
        pltpu.VMEM((CHUNK,), jnp.float32),
        pltpu.VMEM((CHUNK,), jnp.float32),
        pltpu.SemaphoreType.DMA,
        pltpu.SemaphoreType.DMA,
        pltpu.SemaphoreType.DMA,
        pltpu.SemaphoreType.DMA,
    ],
)
def _scan_apply(x_hbm, sums_hbm, wsums_hbm, out_hbm,
                wsums_v, sums_v, offs_v, buf0, buf1,
                isem0, isem1, osem0, osem1):
    wid = lax.axis_index("s") * NC + lax.axis_index("c")
    base = wid * NPW
    bufs = (buf0, buf1)
    isems = (isem0, isem1)
    osems = (osem0, osem1)

    pltpu.sync_copy(wsums_hbm, wsums_v)
    pltpu.sync_copy(sums_hbm.at[pl.ds(wid * SPW, SPW)], sums_v)

    def pre_body(w, cv):
        return cv + plsc.load_gather(
            wsums_v, [jnp.full((L,), 0, jnp.int32) + w * L])

    carry = lax.fori_loop(0, wid, pre_body, jnp.zeros((L,), jnp.float32))

    def off_body(i, cv):
        v = sums_v[pl.ds(i * L, L)]
        offs_v[pl.ds(i * L, L)] = plsc.cumsum(v) - v + cv
        return cv + jnp.sum(v)

    lax.fori_loop(0, SPW // L, off_body, carry, unroll=4)

    idx0 = _idx0()
    ih = [None, None]
    oh = [None, None]
    ih[0] = pltpu.async_copy(x_hbm.at[pl.ds(base, CHUNK)], buf0, isem0)
    for c in range(NCHUNK):
        b = c % 2
        if c + 1 < NCHUNK:
            if oh[1 - b] is not None:
                oh[1 - b].wait()
            ih[1 - b] = pltpu.async_copy(
                x_hbm.at[pl.ds(base + (c + 1) * CHUNK, CHUNK)],
                bufs[1 - b], isems[1 - b])
        ih[b].wait()
        buf = bufs[b]
        offs = tuple(offs_v[pl.ds(c * SPC + g * L, L)] for g in range(GRP))

        @plsc.parallel_loop(0, SEG, unroll=UNROLL, carry=(offs, idx0))
        def t_body(t, ai):
            accs, idxs = ai
            new_accs = []
            for g in range(GRP):
                a = accs[g] + plsc.load_gather(buf, [idxs[g]])
                plsc.store_scatter(buf, [idxs[g]], a)
                new_accs.append(a)
            return tuple(new_accs), tuple(i + 1 for i in idxs)

        del t_body
        oh[b] = pltpu.async_copy(
            buf, out_hbm.at[pl.ds(base + c * CHUNK, CHUNK)], osems[b])

    oh[0].wait()
    oh[1].wait()


def kernel(input_array):
    sums, wsums = _seg_sums(input_array)
    return _scan_apply(input_array, sums, wsums)


# speedup vs baseline: 17.8007x; 3.7378x over previous
import functools

import jax
import jax.numpy as jnp
from jax import lax
from jax.experimental import pallas as pl
from jax.experimental.pallas import tpu as pltpu
from jax.experimental.pallas import tpu_sc as plsc

N = 8388608
NC, NS, L = 2, 16, 16
NW = NC * NS
NPW = N // NW
SEG = 512
GRP = 4
CHUNK = GRP * L * SEG
NCHUNK = NPW // CHUNK
SPC = GRP * L
SPW = NCHUNK * SPC
NSEG = NW * SPW
UNROLL = 4

_mesh = plsc.VectorSubcoreMesh(core_axis_name="c", subcore_axis_name="s")
_params = pltpu.CompilerParams(needs_layout_passes=False)


def _idx0():
    lane = lax.iota(jnp.int32, L)
    return tuple(lane * (SEG - 1) + g * (L * SEG) for g in range(GRP))


@functools.partial(
    pl.kernel,
    mesh=_mesh,
    out_type=(jax.ShapeDtypeStruct((NSEG,), jnp.float32),
              jax.ShapeDtypeStruct((NW * L,), jnp.float32)),
    compiler_params=_params,
    scratch_types=[
        pltpu.VMEM((CHUNK,), jnp.float32),
        pltpu.VMEM((CHUNK,), jnp.float32),
        pltpu.VMEM((SPW,), jnp.float32),
        pltpu.VMEM((L,), jnp.float32),
        pltpu.SemaphoreType.DMA,
        pltpu.SemaphoreType.DMA,
    ],
)
def _seg_sums(x_hbm, sums_hbm, wsums_hbm, buf0, buf1, totals_v, wtot_v,
              sem0, sem1):
    wid = lax.axis_index("s") * NC + lax.axis_index("c")
    base = wid * NPW
    bufs = (buf0, buf1)
    sems = (sem0, sem1)
    idx0 = _idx0()

    handles = [None, None]
    handles[0] = pltpu.async_copy(x_hbm.at[pl.ds(base, CHUNK)], buf0, sem0)
    for c in range(NCHUNK):
        b = c % 2
        if c + 1 < NCHUNK:
            handles[1 - b] = pltpu.async_copy(
                x_hbm.at[pl.ds(base + (c + 1) * CHUNK, CHUNK)],
                bufs[1 - b], sems[1 - b])
        handles[b].wait()
        buf = bufs[b]

        @plsc.parallel_loop(
            0, SEG + L - 1, unroll=UNROLL,
            carry=(tuple(jnp.zeros((L,), jnp.float32) for _ in range(GRP)),
                   idx0, -lax.iota(jnp.int32, L)))
        def t_body(t, ai):
            accs, idxs, d = ai
            mask = (d >= 0) & (d < SEG)
            accs = tuple(
                a + jnp.where(mask, plsc.load_gather(buf, [i], mask=mask),
                              jnp.float32(0.0))
                for a, i in zip(accs, idxs))
            return accs, tuple(i + 1 for i in idxs), d + 1

        accs, _, _ = t_body
        for g in range(GRP):
            totals_v[pl.ds(c * SPC + g * L, L)] = accs[g]

    def tot_body(i, acc):
        return acc + totals_v[pl.ds(i * L, L)]

    tot = lax.fori_loop(0, SPW // L, tot_body, jnp.zeros((L,), jnp.float32),
                        unroll=4)
    wtot_v[...] = jnp.zeros((L,), jnp.float32) + jnp.sum(tot)
    pltpu.sync_copy(totals_v, sums_hbm.at[pl.ds(wid * SPW, SPW)])
    pltpu.sync_copy(wtot_v, wsums_hbm.at[pl.ds(wid * L, L)])


@functools.partial(
    pl.kernel,
    mesh=_mesh,
    out_type=jax.ShapeDtypeStruct((N,), jnp.float32),
    compiler_params=_params,
    scratch_types=[
        pltpu.VMEM((NW * L,), jnp.float32),
        pltpu.VMEM((SPW,), jnp.float32),
        pltpu.VMEM((SPW,), jnp.float32),
        pltpu.VMEM((CHUNK,), jnp.float32),
        pltpu.VMEM((CHUNK,), jnp.float32),
        pltpu.SemaphoreType.DMA,
        pltpu.SemaphoreType.DMA,
        pltpu.SemaphoreType.DMA,
        pltpu.SemaphoreType.DMA,
    ],
)
def _scan_apply(x_hbm, sums_hbm, wsums_hbm, out_hbm,
                wsums_v, sums_v, offs_v, buf0, buf1,
                isem0, isem1, osem0, osem1):
    wid = lax.axis_index("s") * NC + lax.axis_index("c")
    base = wid * NPW
    bufs = (buf0, buf1)
    isems = (isem0, isem1)
    osems = (osem0, osem1)

    pltpu.sync_copy(wsums_hbm, wsums_v)
    pltpu.sync_copy(sums_hbm.at[pl.ds(wid * SPW, SPW)], sums_v)

    def pre_body(w, cv):
        return cv + plsc.load_gather(
            wsums_v, [jnp.full((L,), 0, jnp.int32) + w * L])

    carry = lax.fori_loop(0, wid, pre_body, jnp.zeros((L,), jnp.float32))

    def off_body(i, cv):
        v = sums_v[pl.ds(i * L, L)]
        offs_v[pl.ds(i * L, L)] = plsc.cumsum(v) - v + cv
        return cv + jnp.sum(v)

    lax.fori_loop(0, SPW // L, off_body, carry, unroll=4)

    idx0 = _idx0()
    ih = [None, None]
    oh = [None, None]
    ih[0] = pltpu.async_copy(x_hbm.at[pl.ds(base, CHUNK)], buf0, isem0)
    for c in range(NCHUNK):
        b = c % 2
        if c + 1 < NCHUNK:
            if oh[1 - b] is not None:
                oh[1 - b].wait()
            ih[1 - b] = pltpu.async_copy(
                x_hbm.at[pl.ds(base + (c + 1) * CHUNK, CHUNK)],
                bufs[1 - b], isems[1 - b])
        ih[b].wait()
        buf = bufs[b]
        offs = tuple(offs_v[pl.ds(c * SPC + g * L, L)] for g in range(GRP))

        @plsc.parallel_loop(0, SEG + L - 1, unroll=UNROLL,
                            carry=(offs, idx0, -lax.iota(jnp.int32, L)))
        def t_body(t, ai):
            accs, idxs, d = ai
            mask = (d >= 0) & (d < SEG)
            new_accs = []
            for g in range(GRP):
                v = plsc.load_gather(buf, [idxs[g]], mask=mask)
                a = accs[g] + jnp.where(mask, v, jnp.float32(0.0))
                plsc.store_scatter(buf, [idxs[g]], a, mask=mask)
                new_accs.append(a)
            return tuple(new_accs), tuple(i + 1 for i in idxs), d + 1

        del t_body
        oh[b] = pltpu.async_copy(
            buf, out_hbm.at[pl.ds(base + c * CHUNK, CHUNK)], osems[b])

    oh[0].wait()
    oh[1].wait()


def kernel(input_array):
    sums, wsums = _seg_sums(input_array)
    return _scan_apply(input_array, sums, wsums)
